# baseline (speedup 1.0000x reference)
import functools

import jax
import jax.numpy as jnp
from jax import lax
from jax.experimental import pallas as pl
from jax.experimental.pallas import tpu as pltpu
from jax.experimental.pallas import tpu_sc as plsc

N_ELEC = 128
EMBED_DIM = 256
LANES = 16
NUM_CORES = 1
NUM_SUBCORES = 16
NUM_WORKERS = NUM_CORES * NUM_SUBCORES
ROWS_PER_WORKER = N_ELEC // NUM_WORKERS
CHUNKS = EMBED_DIM // LANES
TYPES_OFF = 2 * EMBED_DIM
AUX_LEN = TYPES_OFF + NUM_WORKERS * LANES


@functools.partial(
    pl.kernel,
    mesh=plsc.VectorSubcoreMesh(
        core_axis_name="c", subcore_axis_name="s", num_cores=NUM_CORES),
    out_type=jax.ShapeDtypeStruct((N_ELEC, EMBED_DIM), jnp.float32),
    scratch_types=[
        pltpu.VMEM((AUX_LEN,), jnp.float32),
        pltpu.VMEM((ROWS_PER_WORKER, EMBED_DIM), jnp.float32),
        pltpu.SemaphoreType.DMA,
    ],
)
def _embed_kernel(aux_hbm, out_hbm, aux_v, rows_v, sem):
    wid = lax.axis_index("s") * NUM_CORES + lax.axis_index("c")
    pltpu.async_copy(aux_hbm, aux_v, sem).wait()
    tv = aux_v[pl.ds(TYPES_OFF + wid * LANES, LANES)]
    half = ROWS_PER_WORKER // 2
    base = wid * ROWS_PER_WORKER
    copies = []
    for h in range(2):
        for c in range(CHUNKS):
            t0 = aux_v[pl.ds(c * LANES, LANES)]
            diff = aux_v[pl.ds(EMBED_DIM + c * LANES, LANES)] - t0
            for r in range(h * half, (h + 1) * half):
                rows_v[r, pl.ds(c * LANES, LANES)] = t0 + tv[r] * diff
        copies.append(pltpu.async_copy(
            rows_v.at[pl.ds(h * half, half)],
            out_hbm.at[pl.ds(base + h * half, half)], sem))
    for cp in copies:
        cp.wait()


def kernel(phys_conf, nucleus_embedding, table, elec_types):
    del phys_conf, nucleus_embedding
    idx = elec_types.reshape(NUM_WORKERS, ROWS_PER_WORKER)
    idx = jnp.pad(idx, ((0, 0), (0, LANES - ROWS_PER_WORKER)))
    aux = jnp.concatenate(
        [table.reshape(-1), idx.astype(jnp.float32).reshape(-1)])
    return _embed_kernel(aux)


# speedup vs baseline: 1.0153x; 1.0153x over previous
import functools

import jax
import jax.numpy as jnp
from jax import lax
from jax.experimental import pallas as pl
from jax.experimental.pallas import tpu as pltpu
from jax.experimental.pallas import tpu_sc as plsc

N_ELEC = 128
EMBED_DIM = 256
LANES = 16
NUM_CORES = 1
NUM_SUBCORES = 16
NUM_WORKERS = NUM_CORES * NUM_SUBCORES
ROWS_PER_WORKER = N_ELEC // NUM_WORKERS
CHUNKS = EMBED_DIM // LANES
TYPES_OFF = 2 * EMBED_DIM
AUX_LEN = TYPES_OFF + NUM_WORKERS * LANES


@functools.partial(
    pl.kernel,
    mesh=plsc.VectorSubcoreMesh(
        core_axis_name="c", subcore_axis_name="s", num_cores=NUM_CORES),
    out_type=jax.ShapeDtypeStruct((N_ELEC, EMBED_DIM), jnp.float32),
    scratch_types=[
        pltpu.VMEM((AUX_LEN,), jnp.float32),
        pltpu.VMEM((ROWS_PER_WORKER, EMBED_DIM), jnp.float32),
        pltpu.SemaphoreType.DMA,
    ],
)
def _embed_kernel(aux_hbm, out_hbm, aux_v, rows_v, sem):
    wid = lax.axis_index("s") * NUM_CORES + lax.axis_index("c")
    pltpu.async_copy(aux_hbm, aux_v, sem).wait()
    tv = aux_v[pl.ds(TYPES_OFF + wid * LANES, LANES)]
    for c in range(CHUNKS):
        t0 = aux_v[pl.ds(c * LANES, LANES)]
        diff = aux_v[pl.ds(EMBED_DIM + c * LANES, LANES)] - t0
        for r in range(ROWS_PER_WORKER):
            rows_v[r, pl.ds(c * LANES, LANES)] = t0 + tv[r] * diff
    pltpu.sync_copy(
        rows_v, out_hbm.at[pl.ds(wid * ROWS_PER_WORKER, ROWS_PER_WORKER)])


def kernel(phys_conf, nucleus_embedding, table, elec_types):
    del phys_conf, nucleus_embedding
    idx = elec_types.reshape(NUM_WORKERS, ROWS_PER_WORKER)
    idx = jnp.pad(idx, ((0, 0), (0, LANES - ROWS_PER_WORKER)))
    aux = jnp.concatenate(
        [table.reshape(-1), idx.astype(jnp.float32).reshape(-1)])
    return _embed_kernel(aux)
